# baseline (speedup 1.0000x reference)
import functools

import jax
import jax.numpy as jnp
from jax import lax
from jax.experimental import pallas as pl
from jax.experimental.pallas import tpu as pltpu
from jax.experimental.pallas import tpu_sc as plsc

_WA = 896
_WB = 128


def _mbuild_body(t_ref, wa_ref, ba_ref, wb_ref, bb_ref, ma_ref, mb_ref):
    dn = (((1,), (1,)), ((), ()))
    ma_ref[...] = (
        lax.dot_general(t_ref[...], wa_ref[...], dn,
                        preferred_element_type=jnp.float32)
        + ba_ref[...]
    )
    mb_ref[...] = (
        lax.dot_general(t_ref[...], wb_ref[...], dn,
                        preferred_element_type=jnp.float32)
        + bb_ref[...]
    )


def _build_tables(table, W, b):
    V, D = W.shape
    wa, ba = W[:_WA], b[:_WA].reshape(1, _WA)
    wb = jnp.pad(W[_WA:], ((0, _WB - (V - _WA)), (0, 0)))
    bb = jnp.pad(b[_WA:], (0, _WB - (V - _WA))).reshape(1, _WB)
    return pl.pallas_call(
        _mbuild_body,
        out_shape=(
            jax.ShapeDtypeStruct((V, _WA), jnp.float32),
            jax.ShapeDtypeStruct((V, _WB), jnp.float32),
        ),
    )(table, wa, ba, wb, bb)


@functools.lru_cache(maxsize=None)
def _make_gather(B, V, CH):
    info = plsc.get_sparse_core_info()
    nc, ns = info.num_cores, info.num_subcores
    nw = nc * ns
    b_per_w = B // nw
    n_ch = b_per_w // CH
    assert B % (CH * nw) == 0 and CH % 8 == 0
    assert n_ch % 2 == 0
    mesh = plsc.VectorSubcoreMesh(core_axis_name="c", subcore_axis_name="s")

    @functools.partial(
        pl.kernel,
        out_type=(
            jax.ShapeDtypeStruct((B, V), jnp.float32),
            jax.ShapeDtypeStruct((B, _WB), jnp.float32),
        ),
        mesh=mesh,
        scratch_types=[
            pltpu.VMEM((b_per_w,), jnp.int32),
            pltpu.VMEM((CH, _WA), jnp.float32),
            pltpu.VMEM((CH, _WA), jnp.float32),
            pltpu.VMEM((CH, _WB), jnp.float32),
            pltpu.VMEM((CH, _WB), jnp.float32),
            pltpu.SemaphoreType.DMA,
            pltpu.SemaphoreType.DMA,
            pltpu.SemaphoreType.DMA,
            pltpu.SemaphoreType.DMA,
        ],
    )
    def gather(ma_hbm, mb_hbm, idx_hbm, out_hbm, t_hbm,
               idx_v, a0, a1, bb0, bb1, sa0, sa1, sb0, sb1):
        wid = lax.axis_index("s") * nc + lax.axis_index("c")
        base = wid * b_per_w
        pltpu.sync_copy(idx_hbm.at[pl.ds(base, b_per_w)], idx_v)

        bufs = ((a0, bb0, sa0, sb0), (a1, bb1, sa1, sb1))

        def start(i, a, bb, sa, sb):
            idxs = idx_v.at[pl.ds(i * CH, CH)]
            pltpu.async_copy(ma_hbm.at[idxs], a, sa)
            pltpu.async_copy(mb_hbm.at[idxs], bb, sb)

        def finish(i, a, bb, sa, sb):
            idxs = idx_v.at[pl.ds(i * CH, CH)]
            pltpu.make_async_copy(ma_hbm.at[idxs], a, sa).wait()
            pltpu.make_async_copy(mb_hbm.at[idxs], bb, sb).wait()
            rows = pl.ds(base + i * CH, CH)
            pltpu.sync_copy(a, out_hbm.at[rows, pl.ds(0, _WA)])
            pltpu.sync_copy(bb, t_hbm.at[rows])

        start(0, *bufs[0])

        def body(j, carry):
            i0 = j * 2
            start(i0 + 1, *bufs[1])
            finish(i0, *bufs[0])

            @pl.when(j < n_ch // 2 - 1)
            def _():
                start(i0 + 2, *bufs[0])

            finish(i0 + 1, *bufs[1])
            return carry

        lax.fori_loop(0, n_ch // 2, body, 0)

    return gather


def _tail_body(seq, o_in_ref, t_ref, o_ref):
    o_ref[...] = t_ref[...].reshape(o_ref.shape)


def _fill_tail(out2d, T, bsz, seq, V, BB):
    out3d = out2d.reshape(bsz, seq, V)
    nlt = V // _WB
    return pl.pallas_call(
        functools.partial(_tail_body, seq),
        grid=(bsz // BB,),
        in_specs=[
            pl.BlockSpec((BB, seq, _WB), lambda i: (i, 0, nlt)),
            pl.BlockSpec((BB * seq, _WB), lambda i: (i, 0)),
        ],
        out_specs=pl.BlockSpec((BB, seq, _WB), lambda i: (i, 0, nlt)),
        out_shape=jax.ShapeDtypeStruct((bsz, seq, V), jnp.float32),
        input_output_aliases={0: 0},
    )(out3d, T)


def kernel(input_seq, emb_table, W, b):
    bsz, seq = input_seq.shape
    v = W.shape[0]
    idx = input_seq.reshape(-1).astype(jnp.int32)
    ma, mb = _build_tables(emb_table, W, b)
    out2d, T = _make_gather(bsz * seq, v, 40)(ma, mb, idx)
    return _fill_tail(out2d, T, bsz, seq, v, 32)


# speedup vs baseline: 1.6173x; 1.6173x over previous
import functools

import jax
import jax.numpy as jnp
from jax import lax
from jax.experimental import pallas as pl
from jax.experimental.pallas import tpu as pltpu
from jax.experimental.pallas import tpu_sc as plsc


@functools.lru_cache(maxsize=None)
def _make_gather(B, D, CH):
    info = plsc.get_sparse_core_info()
    nc, ns = info.num_cores, info.num_subcores
    nw = nc * ns
    b_per_w = B // nw
    n_ch = b_per_w // CH
    assert B % (CH * nw) == 0 and CH % 8 == 0 and CH <= 128
    assert n_ch % 2 == 0
    mesh = plsc.VectorSubcoreMesh(core_axis_name="c", subcore_axis_name="s")

    @functools.partial(
        pl.kernel,
        out_type=jax.ShapeDtypeStruct((B, D), jnp.float32),
        mesh=mesh,
        scratch_types=[
            pltpu.VMEM((b_per_w,), jnp.int32),
            pltpu.VMEM((CH, D), jnp.float32),
            pltpu.VMEM((CH, D), jnp.float32),
            pltpu.SemaphoreType.DMA,
            pltpu.SemaphoreType.DMA,
        ],
    )
    def gather(table_hbm, idx_hbm, out_hbm, idx_v, rows0, rows1, sem0, sem1):
        wid = lax.axis_index("s") * nc + lax.axis_index("c")
        base = wid * b_per_w
        pltpu.sync_copy(idx_hbm.at[pl.ds(base, b_per_w)], idx_v)

        bufs = ((rows0, sem0), (rows1, sem1))

        def start(i, buf, sem):
            pltpu.async_copy(
                table_hbm.at[idx_v.at[pl.ds(i * CH, CH)]], buf, sem
            )

        def finish(i, buf, sem):
            pltpu.make_async_copy(
                table_hbm.at[idx_v.at[pl.ds(i * CH, CH)]], buf, sem
            ).wait()
            pltpu.sync_copy(buf, out_hbm.at[pl.ds(base + i * CH, CH)])

        start(0, *bufs[0])

        def body(j, carry):
            i0 = j * 2
            start(i0 + 1, *bufs[1])
            finish(i0, *bufs[0])

            @pl.when(j < n_ch // 2 - 1)
            def _():
                start(i0 + 2, *bufs[0])

            finish(i0 + 1, *bufs[1])
            return carry

        lax.fori_loop(0, n_ch // 2, body, 0)

    return gather


def _proj_body(bs, e_ref, w_ref, b_ref, o_ref):
    res = (
        lax.dot_general(
            e_ref[...],
            w_ref[...],
            dimension_numbers=(((1,), (1,)), ((), ())),
            preferred_element_type=jnp.float32,
        )
        + b_ref[...]
    )
    o_ref[...] = res.reshape(o_ref.shape)


def _project(E, W, b, bsz, seq, BB):
    B, D = E.shape
    V = W.shape[0]
    return pl.pallas_call(
        functools.partial(_proj_body, seq),
        grid=(bsz // BB,),
        in_specs=[
            pl.BlockSpec((BB * seq, D), lambda i: (i, 0)),
            pl.BlockSpec((V, D), lambda i: (0, 0)),
            pl.BlockSpec((1, V), lambda i: (0, 0)),
        ],
        out_specs=pl.BlockSpec((BB, seq, V), lambda i: (i, 0, 0)),
        out_shape=jax.ShapeDtypeStruct((bsz, seq, V), jnp.float32),
    )(E, W, b.reshape(1, V))


def kernel(input_seq, emb_table, W, b):
    bsz, seq = input_seq.shape
    v = W.shape[0]
    idx = input_seq.reshape(-1).astype(jnp.int32)
    E = _make_gather(bsz * seq, emb_table.shape[1], 80)(emb_table, idx)
    return _project(E, W, b, bsz, seq, 16)
